# baseline (speedup 1.0000x reference)
import jax
import jax.numpy as jnp
from jax.experimental import pallas as pl

_F32 = jnp.float32


def _body(A_r, B_r, C_r, out_r):
    def dot(a, b):
        return jnp.dot(a, b, preferred_element_type=_F32)

    def dgT(a, b):
        return jax.lax.dot_general(a, b, (((1,), (1,)), ((), ())),
                                   preferred_element_type=_F32)

    def rc(shape):
        r = jax.lax.broadcasted_iota(jnp.int32, shape, 0).astype(_F32)
        c = jax.lax.broadcasted_iota(jnp.int32, shape, 1).astype(_F32)
        return r, c

    def fdiv(x, n):
        return jnp.floor((x + 0.5) * (1.0 / n))

    def fmod(x, n):
        return x - n * fdiv(x, n)

    def eq(a, b):
        return (a == b).astype(_F32)

    A = A_r[...]
    B = B_r[...]
    C = C_r[...]

    def rewiden(piece, R0, W0):
        k = W0 // 8
        r1, c1 = rc((R0, R0 * k))
        r2, c2 = rc((8, W0))
        out = None
        for j in range(k):
            Rj = eq(c1, float(k) * r1 + j)
            Cj = eq(c2, r2 + 8.0 * j)
            term = dot(dot(Rj, piece), Cj)
            out = term if out is None else out + term
        return out

    obs0, obs1 = A[0:20], A[20:40]
    ov0, ov1 = A[40:60], A[60:80]
    RESsp = A[80:96]
    tmir = A[96:192]
    tmr = A[192:288]
    sp = rewiden(A[288:336], 16, 24)
    tmic = rewiden(A[336:624], 96, 24)
    tmc = rewiden(A[624:912], 96, 24)
    cqW = rewiden(A[912:1040], 32, 32)
    ckW = rewiden(A[1040:1168], 32, 32)
    rW = rewiden(A[1168:1232], 32, 16)
    emb = rewiden(A[1232:1264], 8, 32)
    r2, c2 = rc((8, 32))
    cqb = sum(dot(A[1264 + j:1265 + j], eq(c2, r2 + 8.0 * j))
              for j in range(4))
    ckb = sum(dot(A[1268 + j:1269 + j], eq(c2, r2 + 8.0 * j))
              for j in range(4))
    rqb = sum(dot(A[1272 + j:1273 + j], eq(c2, r2 + 8.0 * j))
              for j in range(4))

    spi12 = B[0:8]
    bti = B[8:16] + B[16:24]
    bts = B[24:32] + B[32:40]

    spr = C[0:16]
    noise = C[16:36]
    b1 = [C[36:44], C[44:52], C[52:60], C[60:68]]
    r1, c1 = rc((8, 4))
    r2b, c2b = rc((2, 1))
    wcol = []
    for i in range(3):
        piece = C[68 + 4 * i:72 + 4 * i]
        out = None
        for j in range(2):
            Rj = eq(c1, fdiv(r1, 2)) * eq(fmod(r1, 2), j)
            Cj = eq(r2b, j) * eq(c2b, 0)
            term = dot(dot(Rj, piece), Cj)
            out = term if out is None else out + term
        wcol.append(out)
    wi_c, ws_c, wn = wcol

    r, c = rc((24, 8))
    Sel24 = [eq(r, 3.0 * c + d) for d in range(3)]
    r, c = rc((64, 16))
    RS = [eq(fdiv(r, 8), fdiv(c, 2)) * eq(fmod(r, 8), 2.0 * p + fmod(c, 2))
          for p in range(4)]
    r, c = rc((64, 64))
    Kd = [eq(fdiv(r, 8), fdiv(c, 8)) * eq(fmod(c, 8), fmod(r, 8) + (d - 1))
          for d in range(3)]
    r, c = rc((8, 16))
    PLR16 = [eq(c, r + 8.0 * p) for p in range(2)]
    S2 = [eq(c, 2.0 * r + p) for p in range(2)]
    r, c = rc((16, 32))
    PLR32 = [eq(c, r + 16.0 * p) for p in range(2)]
    r, c = rc((8, 32))
    S4 = [eq(c, 4.0 * r + cc) for cc in range(4)]
    r, c = rc((64, 2))
    Q4 = [eq(fmod(r, 8), 2.0 * p + c) for p in range(4)]
    Prow = eq(c, fmod(fdiv(r, 8), 2))
    r, c = rc((8, 8))
    I8 = eq(r, c)
    r, c = rc((2, 2))
    I2 = eq(r, c)
    r, c = rc((2, 8))
    P28 = eq(r, c)
    r, c = rc((20, 400))
    G = eq(20.0 * fdiv(c - r + 20.0, 20) - 20.0, c - r)
    Gb = eq(r, fdiv(c, 20))
    r, c = rc((512, 64))
    C1 = eq(c, fdiv(r, 8))
    Mtt = eq(fmod(r, 8), fmod(c, 8))
    r, c = rc((8, 64))
    C2 = eq(r, fdiv(c, 8))
    r, c = rc((96, 8))
    Rep96 = eq(c, fdiv(r, 12))
    r, c = rc((96, 12))
    Msel96 = eq(c, fmod(r, 12))
    r, c = rc((64, 8))
    R64 = eq(c, fdiv(r, 8))
    ones12 = jnp.ones((12, 1), _F32)
    ones2 = jnp.ones((2, 1), _F32)

    r1, c1 = rc((16, 8))
    r2, c2 = rc((12, 8))
    SPI = []
    for d in range(3):
        out = None
        for o in range(2):
            Ro = eq(c1, fdiv(r1, 2)) * eq(fmod(r1, 2), o)
            Co = eq(r2, 6.0 * o + 3.0 * c2 + d) * (c2 < 2.0).astype(_F32)
            term = dot(dot(Ro, spi12), Co)
            out = term if out is None else out + term
        SPI.append(out)

    X = (dot(obs0, S4[0]) + dot(obs1, S4[1])
         + dot(ov0, S4[2]) + dot(ov1, S4[3]))
    cq = dgT(X, cqW) + cqb
    ck = dgT(X, ckW) + ckb
    logits = dgT(cq, ck)
    iota20 = jax.lax.broadcasted_iota(jnp.int32, (20, 20), 1)
    a = logits
    masks = []
    for _ in range(3):
        mx = jnp.max(a, axis=-1, keepdims=True)
        cand = jnp.where(a >= mx, iota20, 1000000)
        idx = jnp.min(cand, axis=-1, keepdims=True)
        oh = iota20 == idx
        masks.append(oh.astype(_F32))
        a = jnp.where(oh, -jnp.inf, a)

    X2 = dot(ov0, S2[0]) + dot(ov1, S2[1])
    rq = dgT(X2, rW) + rqb
    rlT = dgT(emb, rq)
    iota8 = jax.lax.broadcasted_iota(jnp.int32, (8, 20), 0)
    mx = jnp.max(rlT, axis=0, keepdims=True)
    cand = jnp.where(rlT >= mx, iota8, 1000000)
    ridx = jnp.min(cand, axis=0, keepdims=True)
    rmaskT = (iota8 == ridx).astype(_F32)

    ovT = [dgT(I8, ov0), dgT(I8, ov1)]
    rows = [ovT[0], ovT[1]]
    for m in masks:
        rows.append(dgT(ovT[0], m))
        rows.append(dgT(ovT[1], m))
    combT = jnp.concatenate(rows, axis=0)
    base = dot(combT, G)
    nT = dgT(I2, noise)
    nbase = dot(Prow, dot(nT, Gb))

    RESspi = dot(spr, P28)
    Wtot = jnp.zeros((512, 64), _F32)
    for d in range(3):
        Wd = dot(RS[0], SPI[d]) + dot(RS[2], dot(sp, Sel24[d]))
        if d == 1:
            Wd = Wd + dot(RS[1], RESspi) + dot(RS[3], RESsp)
        Wexp = dot(C1, Wd)
        Wexp = dot(Wexp, C2) * Mtt
        Wtot = Wtot + dot(Wexp, Kd[d])
    bq = jnp.zeros((64, 1), _F32)
    for p in range(4):
        bq = bq + dot(dot(R64, b1[p]) * Q4[p], ones2)
    b512 = dot(C1, bq)

    Wi = [dot(tmic, Sel24[d]) for d in range(3)]
    Ws = [dot(tmc, Sel24[d]) for d in range(3)]
    Wi1 = Wi[1] + tmir
    Ws1 = Ws[1] + tmr
    Mi0 = dot(Wi1, PLR16[0]) + dot(Wi[2], PLR16[1])
    Mi1 = dot(Wi[0], PLR16[0]) + dot(Wi1, PLR16[1])
    Ms0 = dot(Ws1, PLR16[0]) + dot(Ws[2], PLR16[1])
    Ms1 = dot(Ws[0], PLR16[0]) + dot(Ws1, PLR16[1])
    wi96 = dot(Rep96, wi_c)
    ws96 = dot(Rep96, ws_c)
    M0 = dot(wi96 * Mi0, PLR32[0]) + dot(ws96 * Ms0, PLR32[1])
    M1 = dot(wi96 * Mi1, PLR32[0]) + dot(ws96 * Ms1, PLR32[1])
    bcol_i = dot(dot(Rep96, bti) * Msel96, ones12)
    bcol_s = dot(dot(Rep96, bts) * Msel96, ones12)
    btP = wi96 * bcol_i + ws96 * bcol_s

    acc0 = jnp.zeros((12, 400), _F32)
    acc1 = jnp.zeros((12, 400), _F32)
    for r in range(8):
        Xr = base + wn[r:r + 1, 0:1] * nbase
        U = dot(Wtot[64 * r:64 * r + 64], Xr) + b512[64 * r:64 * r + 64]
        Si = jnp.maximum(U[0:16], 0.0) + U[16:32]
        Ss = jnp.maximum(U[32:48], 0.0) + U[48:64]
        Af = jnp.concatenate([Si, Ss], axis=0)
        bb = btP[12 * r:12 * r + 12]
        P0 = dot(M0[12 * r:12 * r + 12], Af) + bb
        P1 = dot(M1[12 * r:12 * r + 12], Af) + bb
        mbn = dot(rmaskT[r:r + 1, :], G)
        acc0 = acc0 + mbn * P0
        acc1 = acc1 + mbn * P1
    out_r[...] = jnp.concatenate([acc0, acc1], axis=0)


def kernel(obs, obs_vel, noise, rule_q_W, rule_q_b, ctx_q_W, ctx_q_b,
           ctx_k_W, ctx_k_b, emb, w_indiv, w_social, w_noise,
           sp_conv_w, sp_conv_b, sp_res_w, sp_res_b,
           tm_conv_w, tm_conv_b, tm_res_w, tm_res_b,
           spi_conv_w, spi_conv_b, spi_res_w, spi_res_b,
           tmi_conv_w, tmi_conv_b, tmi_res_w, tmi_res_b):
    A = jnp.concatenate([
        obs.reshape(40, 8), obs_vel.reshape(40, 8),
        sp_res_w.reshape(16, 8), tmi_res_w.reshape(96, 8),
        tm_res_w.reshape(96, 8), sp_conv_w.reshape(48, 8),
        tmi_conv_w.reshape(288, 8), tm_conv_w.reshape(288, 8),
        ctx_q_W.reshape(128, 8), ctx_k_W.reshape(128, 8),
        rule_q_W.reshape(64, 8), emb.reshape(32, 8),
        ctx_q_b.reshape(4, 8), ctx_k_b.reshape(4, 8),
        rule_q_b.reshape(4, 8)], axis=0)
    B = jnp.concatenate([
        spi_conv_w.reshape(8, 12), tmi_conv_b, tmi_res_b,
        tm_conv_b, tm_res_b], axis=0)
    C = jnp.concatenate([
        spi_res_w.reshape(16, 2), noise.reshape(20, 2),
        spi_conv_b, spi_res_b, sp_conv_b, sp_res_b,
        w_indiv.reshape(4, 2), w_social.reshape(4, 2),
        w_noise.reshape(4, 2)], axis=0)

    acc = pl.pallas_call(
        _body,
        out_shape=jax.ShapeDtypeStruct((24, 400), _F32),
    )(A, B, C)
    return jnp.transpose(acc.reshape(2, 12, 20, 20), (2, 3, 1, 0))


# speedup vs baseline: 2.0030x; 2.0030x over previous
import jax
import jax.numpy as jnp
from jax.experimental import pallas as pl

_F32 = jnp.float32


def _body(obs_r, ov_r, noise_r, rW_r, rqb_r, cqW_r, cqb_r, ckW_r, ckb_r,
          emb_r, wi_r, ws_r, wn_r, sp4_r, spb_r, sr4_r, srb_r, tm4_r,
          tmb_r, tmr4_r, tmrb_r, spi4_r, spib_r, spir4_r, spirb_r,
          tmi4_r, tmib_r, tmir4_r, tmirb_r, out_r):
    def dot(a, b):
        return jnp.dot(a, b, preferred_element_type=_F32)

    def dgT(a, b):
        return jax.lax.dot_general(a, b, (((1,), (1,)), ((), ())),
                                   preferred_element_type=_F32)

    def rc(shape):
        r = jax.lax.broadcasted_iota(jnp.int32, shape, 0).astype(_F32)
        c = jax.lax.broadcasted_iota(jnp.int32, shape, 1).astype(_F32)
        return r, c

    def fdiv(x, n):
        return jnp.floor((x + 0.5) * (1.0 / n))

    def fmod(x, n):
        return x - n * fdiv(x, n)

    def eq(a, b):
        return (a == b).astype(_F32)

    r, c = rc((64, 16))
    RS = [eq(fdiv(r, 8), fdiv(c, 2)) * eq(fmod(r, 8), 2.0 * p + fmod(c, 2))
          for p in range(4)]
    r, c = rc((64, 64))
    Kd = [eq(fdiv(r, 8), fdiv(c, 8)) * eq(fmod(c, 8), fmod(r, 8) + (d - 1))
          for d in range(3)]
    r, c = rc((8, 16))
    PLR16 = [eq(c, r + 8.0 * p) for p in range(2)]
    S2 = [eq(c, 2.0 * r + p) for p in range(2)]
    r, c = rc((16, 32))
    PLR32 = [eq(c, r + 16.0 * p) for p in range(2)]
    r, c = rc((8, 32))
    S4 = [eq(c, 4.0 * r + cc) for cc in range(4)]
    r, c = rc((64, 2))
    Q4 = [eq(fmod(r, 8), 2.0 * p + c) for p in range(4)]
    Prow = eq(c, fmod(fdiv(r, 8), 2))
    r, c = rc((8, 8))
    I8 = eq(r, c)
    r, c = rc((2, 2))
    I2 = eq(r, c)
    r, c = rc((2, 8))
    P28 = eq(r, c)
    r, c = rc((20, 400))
    G = eq(20.0 * fdiv(c - r + 20.0, 20) - 20.0, c - r)
    Gb = eq(r, fdiv(c, 20))
    r, c = rc((512, 64))
    C1 = eq(c, fdiv(r, 8))
    Mtt = eq(fmod(r, 8), fmod(c, 8))
    r, c = rc((8, 64))
    C2 = eq(r, fdiv(c, 8))
    r, c = rc((96, 8))
    Rep96 = eq(c, fdiv(r, 12))
    r, c = rc((96, 12))
    Msel96 = eq(c, fmod(r, 12))
    r, c = rc((64, 8))
    R64 = eq(c, fdiv(r, 8))
    ones12 = jnp.ones((12, 1), _F32)
    ones2 = jnp.ones((2, 1), _F32)

    obs0, obs1 = obs_r[0, 0], obs_r[0, 1]
    ov0, ov1 = ov_r[0, 0], ov_r[0, 1]
    noise = noise_r[...][:, :, 0, 0]

    X = (dot(obs0, S4[0]) + dot(obs1, S4[1])
         + dot(ov0, S4[2]) + dot(ov1, S4[3]))
    cq = dgT(X, cqW_r[...]) + cqb_r[...][None, :]
    ck = dgT(X, ckW_r[...]) + ckb_r[...][None, :]
    logits = dgT(cq, ck)
    iota20 = jax.lax.broadcasted_iota(jnp.int32, (20, 20), 1)
    a = logits
    masks = []
    for _ in range(3):
        mx = jnp.max(a, axis=-1, keepdims=True)
        cand = jnp.where(a >= mx, iota20, 1000000)
        idx = jnp.min(cand, axis=-1, keepdims=True)
        oh = iota20 == idx
        masks.append(oh.astype(_F32))
        a = jnp.where(oh, -jnp.inf, a)

    X2 = dot(ov0, S2[0]) + dot(ov1, S2[1])
    rq = dgT(X2, rW_r[...]) + rqb_r[...][None, :]
    rlT = dgT(emb_r[...], rq)
    iota8 = jax.lax.broadcasted_iota(jnp.int32, (8, 20), 0)
    mx = jnp.max(rlT, axis=0, keepdims=True)
    cand = jnp.where(rlT >= mx, iota8, 1000000)
    ridx = jnp.min(cand, axis=0, keepdims=True)
    rmaskT = (iota8 == ridx).astype(_F32)

    ovT = [dgT(I8, ov0), dgT(I8, ov1)]
    rows = [ovT[0], ovT[1]]
    for m in masks:
        rows.append(dgT(ovT[0], m))
        rows.append(dgT(ovT[1], m))
    combT = jnp.concatenate(rows, axis=0)
    base = dot(combT, G)
    nT = dgT(I2, noise)
    nbase = dot(Prow, dot(nT, Gb))

    w3sp = sp4_r[...].reshape(16, 8, 3)
    w3spi = spi4_r[...].reshape(16, 2, 3)
    RESsp = sr4_r[...].reshape(16, 8, 1)[:, :, 0]
    RESspi = dot(spir4_r[...].reshape(16, 2, 1)[:, :, 0], P28)
    Wtot = jnp.zeros((512, 64), _F32)
    for d in range(3):
        SPI_d = dot(w3spi[:, :, d], P28)
        Wd = dot(RS[0], SPI_d) + dot(RS[2], w3sp[:, :, d])
        if d == 1:
            Wd = Wd + dot(RS[1], RESspi) + dot(RS[3], RESsp)
        Wexp = dot(C1, Wd)
        Wexp = dot(Wexp, C2) * Mtt
        Wtot = Wtot + dot(Wexp, Kd[d])
    b1 = [spib_r[...], spirb_r[...], spb_r[...], srb_r[...]]
    bq = jnp.zeros((64, 1), _F32)
    for p in range(4):
        bq = bq + dot(dot(R64, b1[p]) * Q4[p], ones2)
    b512 = dot(C1, bq)

    w3tmi = tmi4_r[...].reshape(96, 8, 3)
    w3tm = tm4_r[...].reshape(96, 8, 3)
    Wi = [w3tmi[:, :, d] for d in range(3)]
    Ws = [w3tm[:, :, d] for d in range(3)]
    Wi1 = Wi[1] + tmir4_r[...].reshape(96, 8, 1)[:, :, 0]
    Ws1 = Ws[1] + tmr4_r[...].reshape(96, 8, 1)[:, :, 0]
    Mi0 = dot(Wi1, PLR16[0]) + dot(Wi[2], PLR16[1])
    Mi1 = dot(Wi[0], PLR16[0]) + dot(Wi1, PLR16[1])
    Ms0 = dot(Ws1, PLR16[0]) + dot(Ws[2], PLR16[1])
    Ms1 = dot(Ws[0], PLR16[0]) + dot(Ws1, PLR16[1])
    wi96 = dot(Rep96, wi_r[...])
    ws96 = dot(Rep96, ws_r[...])
    M0 = dot(wi96 * Mi0, PLR32[0]) + dot(ws96 * Ms0, PLR32[1])
    M1 = dot(wi96 * Mi1, PLR32[0]) + dot(ws96 * Ms1, PLR32[1])
    bti = tmib_r[...] + tmirb_r[...]
    bts = tmb_r[...] + tmrb_r[...]
    bcol_i = dot(dot(Rep96, bti) * Msel96, ones12)
    bcol_s = dot(dot(Rep96, bts) * Msel96, ones12)
    btP = wi96 * bcol_i + ws96 * bcol_s

    wn = wn_r[...]
    acc0 = jnp.zeros((12, 400), _F32)
    acc1 = jnp.zeros((12, 400), _F32)
    for r in range(8):
        Xr = base + wn[r:r + 1, 0:1] * nbase
        U = dot(Wtot[64 * r:64 * r + 64], Xr) + b512[64 * r:64 * r + 64]
        Si = jnp.maximum(U[0:16], 0.0) + U[16:32]
        Ss = jnp.maximum(U[32:48], 0.0) + U[48:64]
        Af = jnp.concatenate([Si, Ss], axis=0)
        bb = btP[12 * r:12 * r + 12]
        P0 = dot(M0[12 * r:12 * r + 12], Af) + bb
        P1 = dot(M1[12 * r:12 * r + 12], Af) + bb
        mbn = dot(rmaskT[r:r + 1, :], G)
        acc0 = acc0 + mbn * P0
        acc1 = acc1 + mbn * P1
    out_r[...] = jnp.concatenate([acc0, acc1], axis=0)


def kernel(obs, obs_vel, noise, rule_q_W, rule_q_b, ctx_q_W, ctx_q_b,
           ctx_k_W, ctx_k_b, emb, w_indiv, w_social, w_noise,
           sp_conv_w, sp_conv_b, sp_res_w, sp_res_b,
           tm_conv_w, tm_conv_b, tm_res_w, tm_res_b,
           spi_conv_w, spi_conv_b, spi_res_w, spi_res_b,
           tmi_conv_w, tmi_conv_b, tmi_res_w, tmi_res_b):
    acc = pl.pallas_call(
        _body,
        out_shape=jax.ShapeDtypeStruct((24, 400), _F32),
    )(obs, obs_vel, noise, rule_q_W, rule_q_b, ctx_q_W, ctx_q_b,
      ctx_k_W, ctx_k_b, emb, w_indiv, w_social, w_noise,
      sp_conv_w, sp_conv_b, sp_res_w, sp_res_b,
      tm_conv_w, tm_conv_b, tm_res_w, tm_res_b,
      spi_conv_w, spi_conv_b, spi_res_w, spi_res_b,
      tmi_conv_w, tmi_conv_b, tmi_res_w, tmi_res_b)
    return jnp.transpose(acc.reshape(2, 12, 20, 20), (2, 3, 1, 0))


# speedup vs baseline: 2.2550x; 1.1258x over previous
import jax
import jax.numpy as jnp
from jax.experimental import pallas as pl

_F32 = jnp.float32


def _body(obs_r, ov_r, noise_r, cqW_r, ckW_r, cqb_r, ckb_r, rW_r, rqb_r,
          emb_r, wi_r, ws_r, wn_r, spc_r, spr_r, sp_r, sr_r, tmic_r,
          tmir_r, tmc_r, tmr_r, b1a_r, b1b_r, b1c_r, b1d_r, btic_r,
          btir_r, btsc_r, btsr_r, out_r):
    def dot(a, b):
        return jnp.dot(a, b, preferred_element_type=_F32)

    def dgT(a, b):
        return jax.lax.dot_general(a, b, (((1,), (1,)), ((), ())),
                                   preferred_element_type=_F32)

    def rc(shape):
        r = jax.lax.broadcasted_iota(jnp.int32, shape, 0).astype(_F32)
        c = jax.lax.broadcasted_iota(jnp.int32, shape, 1).astype(_F32)
        return r, c

    def fdiv(x, n):
        return jnp.floor((x + 0.5) * (1.0 / n))

    def fmod(x, n):
        return x - n * fdiv(x, n)

    def eq(a, b):
        return (a == b).astype(_F32)

    r, c = rc((8, 32))
    S4 = [eq(c, 4.0 * r + cc) for cc in range(4)]
    r, c = rc((8, 16))
    S2 = [eq(c, 2.0 * r + cc) for cc in range(2)]
    r, c = rc((24, 8))
    Sel24 = [eq(r, 3.0 * c + d) for d in range(3)]
    r, c = rc((6, 8))
    Sel6 = [eq(r, 3.0 * c + d) for d in range(3)]
    r, c = rc((64, 16))
    RS = [eq(fdiv(r, 8), fdiv(c, 2)) * eq(fmod(r, 8), 2.0 * p + fmod(c, 2))
          for p in range(4)]
    r, c = rc((64, 64))
    Kd = [eq(fdiv(r, 8), fdiv(c, 8)) * eq(fmod(c, 8), fmod(r, 8) + (d - 1))
          for d in range(3)]
    r, c = rc((8, 16))
    PLR16 = [eq(c, r + 8.0 * p) for p in range(2)]
    r, c = rc((16, 32))
    PLR32 = [eq(c, r + 16.0 * p) for p in range(2)]
    r, c = rc((64, 2))
    Q4 = [eq(fmod(r, 8), 2.0 * p + c) for p in range(4)]
    Prow = eq(c, fmod(fdiv(r, 8), 2))
    r, c = rc((8, 8))
    I8 = eq(r, c)
    r, c = rc((2, 2))
    I2 = eq(r, c)
    r, c = rc((2, 8))
    P28 = eq(r, c)
    r, c = rc((20, 400))
    G = eq(20.0 * fdiv(c - r + 20.0, 20) - 20.0, c - r)
    Gb = eq(r, fdiv(c, 20))
    r, c = rc((512, 64))
    C1 = eq(c, fdiv(r, 8))
    Mtt = eq(fmod(r, 8), fmod(c, 8))
    r, c = rc((8, 64))
    C2 = eq(r, fdiv(c, 8))
    r, c = rc((96, 8))
    Rep96 = eq(c, fdiv(r, 12))
    r, c = rc((96, 12))
    Msel96 = eq(c, fmod(r, 12))
    r, c = rc((64, 8))
    R64 = eq(c, fdiv(r, 8))
    ones12 = jnp.ones((12, 1), _F32)
    ones2 = jnp.ones((2, 1), _F32)

    obs = obs_r[...]
    ov = ov_r[...]

    X = (dot(obs[0], S4[0]) + dot(obs[1], S4[1])
         + dot(ov[0], S4[2]) + dot(ov[1], S4[3]))
    cq = dgT(X, cqW_r[...]) + cqb_r[...]
    ck = dgT(X, ckW_r[...]) + ckb_r[...]
    logits = dgT(cq, ck)
    iota20 = jax.lax.broadcasted_iota(jnp.int32, (20, 20), 1)
    a = logits
    masks = []
    for _ in range(3):
        mx = jnp.max(a, axis=-1, keepdims=True)
        cand = jnp.where(a >= mx, iota20, 1000000)
        idx = jnp.min(cand, axis=-1, keepdims=True)
        oh = iota20 == idx
        masks.append(oh.astype(_F32))
        a = jnp.where(oh, -jnp.inf, a)

    X2 = dot(ov[0], S2[0]) + dot(ov[1], S2[1])
    rq = dgT(X2, rW_r[...]) + rqb_r[...]
    rlT = dgT(emb_r[...], rq)
    iota8 = jax.lax.broadcasted_iota(jnp.int32, (8, 20), 0)
    mx = jnp.max(rlT, axis=0, keepdims=True)
    cand = jnp.where(rlT >= mx, iota8, 1000000)
    ridx = jnp.min(cand, axis=0, keepdims=True)
    rmaskT = (iota8 == ridx).astype(_F32)

    ovT = [dgT(I8, ov[0]), dgT(I8, ov[1])]
    rows = [ovT[0], ovT[1]]
    for m in masks:
        rows.append(dgT(ovT[0], m))
        rows.append(dgT(ovT[1], m))
    combT = jnp.concatenate(rows, axis=0)
    base = dot(combT, G)
    nT = dgT(I2, noise_r[...])
    nbase = dot(Prow, dot(nT, Gb))

    spc = spc_r[...]
    sp = sp_r[...]
    RESspi = dot(spr_r[...], P28)
    RESsp = sr_r[...]
    Wtot = jnp.zeros((512, 64), _F32)
    for d in range(3):
        Wd = dot(RS[0], dot(spc, Sel6[d])) + dot(RS[2], dot(sp, Sel24[d]))
        if d == 1:
            Wd = Wd + dot(RS[1], RESspi) + dot(RS[3], RESsp)
        Wexp = dot(C1, Wd)
        Wexp = dot(Wexp, C2) * Mtt
        Wtot = Wtot + dot(Wexp, Kd[d])
    bq = jnp.zeros((64, 1), _F32)
    for p, bp_r in enumerate((b1a_r, b1b_r, b1c_r, b1d_r)):
        bq = bq + dot(dot(R64, bp_r[...]) * Q4[p], ones2)
    b512 = dot(C1, bq)

    tmic = tmic_r[...]
    tmc = tmc_r[...]
    Wi = [dot(tmic, Sel24[d]) for d in range(3)]
    Ws = [dot(tmc, Sel24[d]) for d in range(3)]
    Wi1 = Wi[1] + tmir_r[...]
    Ws1 = Ws[1] + tmr_r[...]
    Mi0 = dot(Wi1, PLR16[0]) + dot(Wi[2], PLR16[1])
    Mi1 = dot(Wi[0], PLR16[0]) + dot(Wi1, PLR16[1])
    Ms0 = dot(Ws1, PLR16[0]) + dot(Ws[2], PLR16[1])
    Ms1 = dot(Ws[0], PLR16[0]) + dot(Ws1, PLR16[1])
    wi96 = dot(Rep96, wi_r[...])
    ws96 = dot(Rep96, ws_r[...])
    M0 = dot(wi96 * Mi0, PLR32[0]) + dot(ws96 * Ms0, PLR32[1])
    M1 = dot(wi96 * Mi1, PLR32[0]) + dot(ws96 * Ms1, PLR32[1])
    bti = btic_r[...] + btir_r[...]
    bts = btsc_r[...] + btsr_r[...]
    bcol_i = dot(dot(Rep96, bti) * Msel96, ones12)
    bcol_s = dot(dot(Rep96, bts) * Msel96, ones12)
    btP = wi96 * bcol_i + ws96 * bcol_s

    wn = wn_r[...]
    wn512 = dot(C1, dot(R64, wn))
    U_all = (dot(Wtot, base) + wn512 * dot(Wtot, nbase)
             + b512)
    rmG = dot(rmaskT, G)
    acc0 = jnp.zeros((12, 400), _F32)
    acc1 = jnp.zeros((12, 400), _F32)
    for r in range(8):
        U = U_all[64 * r:64 * r + 64]
        Si = jnp.maximum(U[0:16], 0.0) + U[16:32]
        Ss = jnp.maximum(U[32:48], 0.0) + U[48:64]
        A = jnp.concatenate([Si, Ss], axis=0)
        bb = btP[12 * r:12 * r + 12]
        P0 = dot(M0[12 * r:12 * r + 12], A) + bb
        P1 = dot(M1[12 * r:12 * r + 12], A) + bb
        mbn = rmG[r:r + 1, :]
        acc0 = acc0 + mbn * P0
        acc1 = acc1 + mbn * P1
    out_r[...] = jnp.concatenate([acc0, acc1], axis=0)


def kernel(obs, obs_vel, noise, rule_q_W, rule_q_b, ctx_q_W, ctx_q_b,
           ctx_k_W, ctx_k_b, emb, w_indiv, w_social, w_noise,
           sp_conv_w, sp_conv_b, sp_res_w, sp_res_b,
           tm_conv_w, tm_conv_b, tm_res_w, tm_res_b,
           spi_conv_w, spi_conv_b, spi_res_w, spi_res_b,
           tmi_conv_w, tmi_conv_b, tmi_res_w, tmi_res_b):
    acc = pl.pallas_call(
        _body,
        out_shape=jax.ShapeDtypeStruct((24, 400), _F32),
    )(obs[0], obs_vel[0], noise.reshape(20, 2),
      ctx_q_W, ctx_k_W, ctx_q_b[None, :], ctx_k_b[None, :],
      rule_q_W, rule_q_b[None, :], emb,
      w_indiv, w_social, w_noise,
      spi_conv_w.reshape(16, 6), spi_res_w.reshape(16, 2),
      sp_conv_w.reshape(16, 24), sp_res_w.reshape(16, 8),
      tmi_conv_w.reshape(96, 24), tmi_res_w.reshape(96, 8),
      tm_conv_w.reshape(96, 24), tm_res_w.reshape(96, 8),
      spi_conv_b, spi_res_b, sp_conv_b, sp_res_b,
      tmi_conv_b, tmi_res_b, tm_conv_b, tm_res_b)
    return jnp.transpose(acc.reshape(2, 12, 20, 20), (2, 3, 1, 0))
